# Initial kernel scaffold; baseline (speedup 1.0000x reference)
import jax
import jax.numpy as jnp
from jax.experimental import pallas as pl


def kernel(x, edge_index, batch, edge_attr, mode, W1, b1, W2, b2, fc1_W, fc1_b, gamma, beta, fc3_W, fc3_b):
    raise NotImplementedError("write your pallas kernel here")


# speedup vs baseline: 10.9661x; 10.9661x over previous
import functools

import jax
import jax.numpy as jnp
from jax import lax
from jax.experimental import pallas as pl
from jax.experimental.pallas import tpu as pltpu
from jax.experimental.pallas import tpu_sc as plsc

NC = 2
NS = 16
NW = NC * NS
L = 16

N = 10000
E = 320000
G = 64
F = 128
EPT = E // NW
CH = 128
NFULL = EPT // CH
TAIL = EPT - NFULL * CH
ZRS = 632


def _mesh():
    return plsc.VectorSubcoreMesh(
        core_axis_name="c", subcore_axis_name="s", num_cores=NC, num_subcores=NS
    )


_SC_PARAMS = pltpu.CompilerParams(needs_layout_passes=False)

_DOT_KW = dict(preferred_element_type=jnp.float32)


@functools.partial(
    pl.kernel,
    out_type=jax.ShapeDtypeStruct((NC, N), jnp.float32),
    mesh=_mesh(),
    compiler_params=_SC_PARAMS,
    scratch_types=[
        pltpu.VMEM((1, CH), jnp.int32),
        pltpu.VMEM((1, CH), jnp.float32),
        pltpu.VMEM((2000,), jnp.float32),
        pltpu.VMEM_SHARED((N,), jnp.float32),
    ],
)
def _deg_kernel(dst_hbm, w_hbm, out_hbm, dst_v, w_v, zbuf, deg_sh):
    c = lax.axis_index("c")
    s = lax.axis_index("s")
    wid = s * NC + c
    ebase = wid * EPT

    def zt(i, _):
        zbuf[pl.ds(i * L, L)] = jnp.zeros((L,), jnp.float32)
        return 0

    lax.fori_loop(0, 2000 // L, zt, 0)

    @pl.when(s < 5)
    def _():
        pltpu.sync_copy(zbuf, deg_sh.at[pl.ds(s * 2000, 2000)])

    plsc.subcore_barrier()

    def chunk(j, _):
        base = ebase + j * CH
        pltpu.sync_copy(dst_hbm.at[pl.ds(base, CH)], dst_v.at[0])
        pltpu.sync_copy(w_hbm.at[pl.ds(base, CH)], w_v.at[0])
        pltpu.sync_copy(w_v.at[0], deg_sh.at[dst_v.at[0]], add=True)
        return 0

    lax.fori_loop(0, NFULL, chunk, 0)

    tbase = ebase + NFULL * CH
    pltpu.sync_copy(dst_hbm.at[pl.ds(tbase, TAIL)], dst_v.at[0, pl.ds(0, TAIL)])
    pltpu.sync_copy(w_hbm.at[pl.ds(tbase, TAIL)], w_v.at[0, pl.ds(0, TAIL)])
    pltpu.sync_copy(w_v.at[0, pl.ds(0, TAIL)],
                    deg_sh.at[dst_v.at[0, pl.ds(0, TAIL)]], add=True)

    plsc.subcore_barrier()

    @pl.when(s == 0)
    def _():
        pltpu.sync_copy(deg_sh, out_hbm.at[c])


@functools.partial(
    pl.kernel,
    out_type=jax.ShapeDtypeStruct((NC * N, F), jnp.float32),
    mesh=_mesh(),
    compiler_params=_SC_PARAMS,
    scratch_types=[
        pltpu.VMEM((1, CH), jnp.int32),
        pltpu.VMEM((1, CH), jnp.int32),
        pltpu.VMEM((1, TAIL), jnp.int32),
        pltpu.VMEM((1, TAIL), jnp.int32),
        pltpu.VMEM((CH,), jnp.float32),
        pltpu.VMEM((CH, F), jnp.float32),
        pltpu.VMEM_SHARED((N, F), jnp.float32),
        pltpu.SemaphoreType.DMA,
    ],
)
def _spmm_kernel(y_hbm, src_hbm, dst_hbm, w_hbm, out_hbm,
                 src_v, dst_v, src_t, dst_t, w_v, rows_v, acc_sh, sem):
    c = lax.axis_index("c")
    s = lax.axis_index("s")
    wid = s * NC + c
    ebase = wid * EPT

    def zt_all(i, _):
        k = i // (F // L)
        o = lax.rem(i, F // L)
        rows_v[k, pl.ds(o * L, L)] = jnp.zeros((L,), jnp.float32)
        return 0

    lax.fori_loop(0, CH * (F // L), zt_all, 0)

    myrows = jnp.where(s == NS - 1, N - (NS - 1) * ZRS, ZRS)
    rbase = s * ZRS

    def zcopy(i, _):
        pltpu.sync_copy(rows_v.at[pl.ds(0, 8)],
                        acc_sh.at[pl.ds(rbase + i * 8, 8)])
        return 0

    lax.fori_loop(0, myrows // 8, zcopy, 0)

    plsc.subcore_barrier()

    def do_edges(base, n_edges, src_b, dst_b):
        pltpu.sync_copy(src_hbm.at[pl.ds(base, n_edges)], src_b.at[0])
        pltpu.sync_copy(dst_hbm.at[pl.ds(base, n_edges)], dst_b.at[0])
        pltpu.sync_copy(w_hbm.at[pl.ds(base, n_edges)],
                        w_v.at[pl.ds(0, n_edges)])
        pltpu.async_copy(y_hbm.at[src_b.at[0]],
                         rows_v.at[pl.ds(0, n_edges)], sem).wait()

        dn = lax.GatherDimensionNumbers(
            offset_dims=(), collapsed_slice_dims=(0,), start_index_map=(0,))

        def grp(g, _):
            e0 = g * L
            wv = w_v[pl.ds(e0, L)]
            for j in range(L):
                wj = lax.gather(wv, jnp.full((L, 1), j, jnp.int32), dn,
                                slice_sizes=(1,),
                                mode=lax.GatherScatterMode.PROMISE_IN_BOUNDS)
                for k in range(F // L):
                    sl = pl.ds(k * L, L)
                    rows_v[e0 + j, sl] = rows_v[e0 + j, sl] * wj
            return 0

        lax.fori_loop(0, n_edges // L, grp, 0)
        pltpu.sync_copy(rows_v.at[pl.ds(0, n_edges)],
                        acc_sh.at[dst_b.at[0]], add=True)

    def chunk(j, _):
        do_edges(ebase + j * CH, CH, src_v, dst_v)
        return 0

    lax.fori_loop(0, NFULL, chunk, 0)
    do_edges(ebase + NFULL * CH, TAIL, src_t, dst_t)

    plsc.subcore_barrier()

    def dcopy(i, _):
        pltpu.sync_copy(acc_sh.at[pl.ds(rbase + i * 8, 8)],
                        out_hbm.at[pl.ds(c * N + rbase + i * 8, 8)])
        return 0

    lax.fori_loop(0, myrows // 8, dcopy, 0)


def _tc_a_body(x_ref, degp_ref, w1_ref, xw1_ref, y1_ref, dis_ref):
    deg = degp_ref[0] + degp_ref[1] + 1.0
    dis = jnp.where(deg > 0, lax.rsqrt(deg), 0.0)
    dis_ref[...] = dis
    xw1 = lax.dot_general(x_ref[...], w1_ref[...], (((1,), (1,)), ((), ())),
                          **_DOT_KW)
    xw1_ref[...] = xw1
    y1_ref[...] = xw1 * dis


def _tc_b_body(accp_ref, xw1_ref, dis_ref, w2_ref, b1_ref, xw2_ref, y2_ref):
    dis = dis_ref[...]
    acc = accp_ref[0] + accp_ref[1]
    h = jnp.maximum(
        acc * dis + xw1_ref[...] * (dis * dis) + b1_ref[...][None, :],
        0.0,
    )
    xw2 = lax.dot_general(h, w2_ref[...], (((1,), (1,)), ((), ())), **_DOT_KW)
    xw2_ref[...] = xw2
    y2_ref[...] = xw2 * dis


def _tc_c_body(accp_ref, xw2_ref, dis_ref, b2_ref, batch_ref,
               fc1w_ref, fc1b_ref, gamma_ref, beta_ref, fc3w_ref, fc3b_ref,
               out_ref):
    dis = dis_ref[...]
    acc = accp_ref[0] + accp_ref[1]
    h = jnp.maximum(
        acc * dis + xw2_ref[...] * (dis * dis) + b2_ref[...][None, :],
        0.0,
    )
    gids = lax.broadcasted_iota(jnp.int32, (N, G), 1)
    onehot = (batch_ref[...] == gids).astype(jnp.float32)
    cnt = lax.dot_general(onehot, jnp.ones((N, 1), jnp.float32),
                          (((0,), (0,)), ((), ())),
                          preferred_element_type=jnp.float32,
                          precision=lax.Precision.HIGHEST)
    pooled = lax.dot_general(onehot, h, (((0,), (0,)), ((), ())),
                             preferred_element_type=jnp.float32,
                             precision=lax.Precision.HIGHEST)
    pooled = pooled / jnp.maximum(cnt, 1.0)
    z = jnp.maximum(
        lax.dot_general(pooled, fc1w_ref[...], (((1,), (1,)), ((), ())),
                        **_DOT_KW)
        + fc1b_ref[...][None, :],
        0.0,
    )
    mu = jnp.mean(z, axis=0)
    d = z - mu[None, :]
    var = jnp.mean(d * d, axis=0)
    zn = d * lax.rsqrt(var + 1e-5)[None, :] * gamma_ref[...][None, :] \
        + beta_ref[...][None, :]
    outp = lax.dot_general(zn, fc3w_ref[...], (((1,), (1,)), ((), ())),
                           **_DOT_KW)
    out_ref[...] = outp[:, 0:1] + fc3b_ref[0]


def _tc_a(x, deg_part, w1):
    return pl.pallas_call(
        _tc_a_body,
        out_shape=(
            jax.ShapeDtypeStruct((N, F), jnp.float32),
            jax.ShapeDtypeStruct((N, F), jnp.float32),
            jax.ShapeDtypeStruct((N, 1), jnp.float32),
        ),
    )(x, deg_part, w1)


def _tc_b(accp, xw1, dis, w2, b1):
    return pl.pallas_call(
        _tc_b_body,
        out_shape=(
            jax.ShapeDtypeStruct((N, F), jnp.float32),
            jax.ShapeDtypeStruct((N, F), jnp.float32),
        ),
    )(accp, xw1, dis, w2, b1)


def _tc_c(accp, xw2, dis, b2, batch, fc1_W, fc1_b, gamma, beta, fc3_W, fc3_b):
    return pl.pallas_call(
        _tc_c_body,
        in_specs=[pl.BlockSpec(memory_space=pltpu.VMEM)] * 10
        + [pl.BlockSpec(memory_space=pltpu.SMEM)],
        out_specs=pl.BlockSpec(memory_space=pltpu.VMEM),
        out_shape=jax.ShapeDtypeStruct((G, 1), jnp.float32),
    )(accp, xw2, dis, b2, batch, fc1_W, fc1_b, gamma, beta, fc3_W, fc3_b)


def kernel(x, edge_index, batch, edge_attr, mode, W1, b1, W2, b2,
           fc1_W, fc1_b, gamma, beta, fc3_W, fc3_b):
    src = edge_index[0]
    dst = edge_index[1]

    deg_part = _deg_kernel(dst, edge_attr).reshape(NC, N, 1)
    xw1, y1, dis = _tc_a(x, deg_part, W1)
    acc1 = _spmm_kernel(y1, src, dst, edge_attr).reshape(NC, N, F)
    xw2, y2 = _tc_b(acc1, xw1, dis, W2, b1)
    acc2 = _spmm_kernel(y2, src, dst, edge_attr).reshape(NC, N, F)
    fc3_Wp = jnp.concatenate(
        [fc3_W, jnp.zeros((7, fc3_W.shape[1]), fc3_W.dtype)], axis=0)
    return _tc_c(acc2, xw2, dis, b2, batch.reshape(N, 1),
                 fc1_W, fc1_b, gamma, beta, fc3_Wp, fc3_b)
